# Initial kernel scaffold; baseline (speedup 1.0000x reference)
import jax
import jax.numpy as jnp
from jax.experimental import pallas as pl


def kernel(x, edge_index, edge_label_index, W1, a1s, a1d, b1, W2, a2s, a2d, b2, Wp1, bp1, g1, be1, Wp2, bp2, g2, be2, Wp3, bp3):
    raise NotImplementedError("write your pallas kernel here")


# speedup vs baseline: 1.1013x; 1.1013x over previous
import functools

import jax
import jax.numpy as jnp
from jax.experimental import pallas as pl

N = 10000
F_IN = 128
HID = 256
OUT = 128
HEADS = 4


def _mm_kernel(x_ref, w_ref, o_ref):
    o_ref[...] = jnp.dot(x_ref[...], w_ref[...],
                         preferred_element_type=jnp.float32)


def _matmul(x, w, bm=1000):
    m, k = x.shape
    _, n = w.shape
    return pl.pallas_call(
        _mm_kernel,
        grid=(m // bm,),
        in_specs=[pl.BlockSpec((bm, k), lambda i: (i, 0)),
                  pl.BlockSpec((k, n), lambda i: (0, 0))],
        out_specs=pl.BlockSpec((bm, n), lambda i: (i, 0)),
        out_shape=jax.ShapeDtypeStruct((m, n), jnp.float32),
    )(x, w)


def _gat(x, src, dst, W, a_s, a_d, b, heads, C, concat, n):
    h = _matmul(x, W).reshape(n, heads, C)
    es = jnp.sum(h * a_s[None], axis=-1)
    ed = jnp.sum(h * a_d[None], axis=-1)
    e = jax.nn.leaky_relu(es[src] + ed[dst], 0.2)
    p = jnp.exp(e)
    ssum = jax.ops.segment_sum(p, dst, num_segments=n)
    alpha = p / (ssum[dst] + 1e-16)
    out = jax.ops.segment_sum(h[src] * alpha[:, :, None], dst, num_segments=n)
    if concat:
        out = out.reshape(n, heads * C)
    else:
        out = out.mean(axis=1)
    return out + b


def kernel(x, edge_index, edge_label_index, W1, a1s, a1d, b1, W2, a2s, a2d,
           b2, Wp1, bp1, g1, be1, Wp2, bp2, g2, be2, Wp3, bp3):
    n = x.shape[0]
    src, dst = edge_index[0], edge_index[1]
    z = _gat(x, src, dst, W1, a1s, a1d, b1, HEADS, HID, True, n)
    z = jax.nn.elu(z)
    z = _gat(z, src, dst, W2, a2s, a2d, b2, 1, OUT, False, n)
    row, col = edge_label_index[0], edge_label_index[1]
    ef = jnp.concatenate([z[row], z[col]], axis=-1)
    bn_scale = 1.0 / jnp.sqrt(1.0 + 1e-5)
    h = jax.nn.relu((ef @ Wp1 + bp1) * bn_scale * g1 + be1)
    h = jax.nn.relu((h @ Wp2 + bp2) * bn_scale * g2 + be2)
    out = (h @ Wp3 + bp3).squeeze(-1)
    return out


# speedup vs baseline: 9.8618x; 8.9544x over previous
import functools

import jax
import jax.numpy as jnp
from jax import lax
from jax.experimental import pallas as pl
from jax.experimental.pallas import tpu as pltpu
from jax.experimental.pallas import tpu_sc as plsc

N = 10000
F_IN = 128
HID = 256
OUT = 128
HEADS = 4
NPAD = 10240
NW = 32
EB = 80


def _mm_kernel(x_ref, w_ref, o_ref):
    o_ref[...] = jnp.dot(x_ref[...], w_ref[...],
                         preferred_element_type=jnp.float32)


def _matmul(x, w, bm=1000):
    m, k = x.shape
    _, n = w.shape
    return pl.pallas_call(
        _mm_kernel,
        grid=(m // bm,),
        in_specs=[pl.BlockSpec((bm, k), lambda i: (i, 0)),
                  pl.BlockSpec((k, n), lambda i: (0, 0))],
        out_specs=pl.BlockSpec((bm, n), lambda i: (i, 0)),
        out_shape=jax.ShapeDtypeStruct((m, n), jnp.float32),
    )(x, w)


def _edge_scores(esed, src, dst, nh, n):
    e = src.shape[0]
    per_w = e // NW
    nblk = per_w // EB
    nss = NPAD * nh
    csz = nss // 16
    zeros = jnp.zeros((nss,), jnp.float32)
    mesh = plsc.VectorSubcoreMesh(core_axis_name="c", subcore_axis_name="s")

    @functools.partial(
        pl.kernel, mesh=mesh,
        out_type=[jax.ShapeDtypeStruct((nh * e,), jnp.float32),
                  jax.ShapeDtypeStruct((2 * nss,), jnp.float32)],
        scratch_types=[pltpu.VMEM((EB,), jnp.int32),
                       pltpu.VMEM((EB,), jnp.int32),
                       pltpu.VMEM((EB,), jnp.int32),
                       pltpu.VMEM((EB,), jnp.float32),
                       pltpu.VMEM((EB,), jnp.float32),
                       pltpu.VMEM((nh * EB,), jnp.float32),
                       pltpu.VMEM((EB,), jnp.int32),
                       pltpu.VMEM_SHARED((nss,), jnp.float32),
                       pltpu.SemaphoreType.DMA],
    )
    def k(esed_h, src_h, dst_h, zero_h, pt_h, ssum_h,
          srcv, dstv, idxg, gsv, gdv, pbuf, idxb, sh_ssum, sem):
        c = lax.axis_index("c")
        s = lax.axis_index("s")
        wid = s * 2 + c
        ebase = wid * per_w
        pltpu.sync_copy(zero_h.at[pl.ds(s * csz, csz)],
                        sh_ssum.at[pl.ds(s * csz, csz)])
        plsc.subcore_barrier()

        def blk(ib, carry):
            eoff = ebase + ib * EB
            bofs = (eoff // EB) * (nh * EB)
            pltpu.sync_copy(src_h.at[pl.ds(eoff, EB)], srcv)
            pltpu.sync_copy(dst_h.at[pl.ds(eoff, EB)], dstv)
            for h in range(nh):
                for i in range(EB // 16):
                    sl = pl.ds(i * 16, 16)
                    idxg[sl] = srcv[sl] + (h * n)
                pltpu.async_copy(esed_h.at[idxg], gsv, sem).wait()
                for i in range(EB // 16):
                    sl = pl.ds(i * 16, 16)
                    idxg[sl] = dstv[sl] + ((nh + h) * n)
                pltpu.async_copy(esed_h.at[idxg], gdv, sem).wait()
                for i in range(EB // 16):
                    sl = pl.ds(i * 16, 16)
                    q = gsv[sl] + gdv[sl]
                    p = jnp.exp(jnp.maximum(q, 0.2 * q))
                    pbuf[pl.ds(h * EB + i * 16, 16)] = p
                    idxb[sl] = dstv[sl] * nh + h
                pltpu.sync_copy(pbuf.at[pl.ds(h * EB, EB)],
                                sh_ssum.at[idxb], add=True)
            pltpu.sync_copy(pbuf, pt_h.at[pl.ds(bofs, nh * EB)])
            return carry

        lax.fori_loop(0, nblk, blk, 0)
        plsc.subcore_barrier()
        pltpu.sync_copy(sh_ssum.at[pl.ds(s * csz, csz)],
                        ssum_h.at[pl.ds(c * nss + s * csz, csz)])

    return k(esed, src, dst, zeros)


def _scatter_msgs(table, pt, src, dst, nc, nh):
    e = src.shape[0]
    cw = table.shape[2]
    per_w = e // NW
    nblk = per_w // EB
    csz = NPAD // 16
    zeros = jnp.zeros((NPAD, cw), jnp.float32)
    mesh = plsc.VectorSubcoreMesh(core_axis_name="c", subcore_axis_name="s")

    nhq = pt.shape[0] // e

    @functools.partial(
        pl.kernel, mesh=mesh,
        out_type=jax.ShapeDtypeStruct((nc * 2 * NPAD, cw), jnp.float32),
        scratch_types=[pltpu.VMEM((EB,), jnp.int32),
                       pltpu.VMEM((EB,), jnp.int32),
                       pltpu.VMEM((EB,), jnp.float32),
                       pltpu.VMEM((EB, cw), jnp.float32),
                       pltpu.VMEM_SHARED((NPAD, cw), jnp.float32),
                       pltpu.SemaphoreType.DMA],
    )
    def k(tab_h, pt_h, src_h, dst_h, zero_h, acc_h,
          srcv, dstv, pv, rows, sh_acc, sem):
        c = lax.axis_index("c")
        s = lax.axis_index("s")
        ebase = c * (e // 2) + s * per_w
        for kk in range(nc):
            hk = kk // (nc // nh)
            pltpu.sync_copy(zero_h.at[pl.ds(s * csz, csz)],
                            sh_acc.at[pl.ds(s * csz, csz)])
            plsc.subcore_barrier()

            def blk(ib, carry):
                eoff = ebase + ib * EB
                pltpu.sync_copy(src_h.at[pl.ds(eoff, EB)], srcv)
                pltpu.sync_copy(dst_h.at[pl.ds(eoff, EB)], dstv)
                pvoff = (eoff // EB) * (nhq * EB) + hk * EB
                pltpu.sync_copy(pt_h.at[pl.ds(pvoff, EB)], pv)
                pltpu.async_copy(tab_h.at[kk].at[srcv], rows, sem).wait()

                def grp(g, cc):
                    pg = pv[pl.ds(g * 16, 16)]
                    for e16 in range(16):
                        pe = pg[e16]
                        i = g * 16 + e16
                        for j in range(cw // 16):
                            rows[i, pl.ds(j * 16, 16)] = (
                                rows[i, pl.ds(j * 16, 16)] * pe)
                    return cc

                lax.fori_loop(0, EB // 16, grp, 0)
                pltpu.sync_copy(rows, sh_acc.at[dstv], add=True)
                return carry

            lax.fori_loop(0, nblk, blk, 0)
            plsc.subcore_barrier()
            pltpu.sync_copy(
                sh_acc.at[pl.ds(s * csz, csz)],
                acc_h.at[pl.ds((kk * 2 + c) * NPAD + s * csz, csz)])
            plsc.subcore_barrier()

    return k(table, pt, src, dst, zeros)


def _pair_gather(z, row, col):
    eq = row.shape[0]
    d = z.shape[1]
    per_w = eq // NW
    blk = 128
    mesh = plsc.VectorSubcoreMesh(core_axis_name="c", subcore_axis_name="s")

    @functools.partial(
        pl.kernel, mesh=mesh,
        out_type=[jax.ShapeDtypeStruct((eq, d), jnp.float32),
                  jax.ShapeDtypeStruct((eq, d), jnp.float32)],
        scratch_types=[pltpu.VMEM((blk,), jnp.int32),
                       pltpu.VMEM((blk, d), jnp.float32),
                       pltpu.SemaphoreType.DMA],
    )
    def k(z_h, row_h, col_h, zr_h, zc_h, idxv, rows, sem):
        wid = lax.axis_index("s") * 2 + lax.axis_index("c")
        base = wid * per_w

        def body(i, carry):
            off = base + i * blk
            pltpu.sync_copy(row_h.at[pl.ds(off, blk)], idxv)
            pltpu.async_copy(z_h.at[idxv], rows, sem).wait()
            pltpu.sync_copy(rows, zr_h.at[pl.ds(off, blk)])
            pltpu.sync_copy(col_h.at[pl.ds(off, blk)], idxv)
            pltpu.async_copy(z_h.at[idxv], rows, sem).wait()
            pltpu.sync_copy(rows, zc_h.at[pl.ds(off, blk)])
            return carry

        lax.fori_loop(0, per_w // blk, body, 0)

    return k(z, row, col)


def kernel(x, edge_index, edge_label_index, W1, a1s, a1d, b1, W2, a2s, a2d,
           b2, Wp1, bp1, g1, be1, Wp2, bp2, g2, be2, Wp3, bp3):
    n = x.shape[0]
    src = edge_index[0].astype(jnp.int32)
    dst = edge_index[1].astype(jnp.int32)

    h1 = _matmul(x, W1)
    h1h = h1.reshape(n, HEADS, HID)
    es1 = jnp.sum(h1h * a1s[None], axis=-1)
    ed1 = jnp.sum(h1h * a1d[None], axis=-1)
    esed1 = jnp.concatenate([es1.T.ravel(), ed1.T.ravel()])
    pt1, ssum1p = _edge_scores(esed1, src, dst, HEADS, n)
    ssum1 = ssum1p.reshape(2, -1).sum(0)[:n * HEADS].reshape(n, HEADS)
    table1 = h1.reshape(n, 8, 128).transpose(1, 0, 2)
    acc1 = _scatter_msgs(table1, pt1, src, dst, 8, HEADS)
    acc1 = acc1.reshape(8, 2, NPAD, 128).sum(1)[:, :n]
    out1 = acc1.transpose(1, 0, 2).reshape(n, HEADS, HID)
    out1 = out1 / (ssum1[:, :, None] + 1e-16)
    z = jax.nn.elu(out1.reshape(n, HEADS * HID) + b1)

    h2 = _matmul(z, W2)
    es2 = h2 @ a2s[0]
    ed2 = h2 @ a2d[0]
    esed2 = jnp.concatenate([es2, ed2])
    pt2, ssum2p = _edge_scores(esed2, src, dst, 1, n)
    ssum2 = ssum2p.reshape(2, -1).sum(0)[:n]
    acc2 = _scatter_msgs(h2[None], pt2, src, dst, 1, 1)
    acc2 = acc2.reshape(1, 2, NPAD, 128).sum(1)[0, :n]
    z2 = acc2 / (ssum2[:, None] + 1e-16) + b2

    row = edge_label_index[0].astype(jnp.int32)
    col = edge_label_index[1].astype(jnp.int32)
    zr, zc = _pair_gather(z2, row, col)
    ef = jnp.concatenate([zr, zc], axis=-1)
    bn_scale = 1.0 / jnp.sqrt(1.0 + 1e-5)
    h = jax.nn.relu((ef @ Wp1 + bp1) * bn_scale * g1 + be1)
    h = jax.nn.relu((h @ Wp2 + bp2) * bn_scale * g2 + be2)
    out = (h @ Wp3 + bp3).squeeze(-1)
    return out


# speedup vs baseline: 11.9507x; 1.2118x over previous
import functools

import jax
import jax.numpy as jnp
from jax import lax
from jax.experimental import pallas as pl
from jax.experimental.pallas import tpu as pltpu
from jax.experimental.pallas import tpu_sc as plsc

N = 10000
F_IN = 128
HID = 256
OUT = 128
HEADS = 4
NPAD = 10240
NW = 32
EB = 128
EPAD = 327680

_SC_PARAMS = pltpu.CompilerParams(needs_layout_passes=False)


def _mm_kernel(x_ref, w_ref, o_ref):
    o_ref[...] = jnp.dot(x_ref[...], w_ref[...],
                         preferred_element_type=jnp.float32)


def _matmul(x, w, bm=1000):
    m, k = x.shape
    _, n = w.shape
    return pl.pallas_call(
        _mm_kernel,
        grid=(m // bm,),
        in_specs=[pl.BlockSpec((bm, k), lambda i: (i, 0)),
                  pl.BlockSpec((k, n), lambda i: (0, 0))],
        out_specs=pl.BlockSpec((bm, n), lambda i: (i, 0)),
        out_shape=jax.ShapeDtypeStruct((m, n), jnp.float32),
    )(x, w)


def _gat_layer(table, esed, src2, dst2, nc, nh, e_real):
    n = table.shape[1]
    cw = table.shape[2]
    per_w = EPAD // NW
    nblk = per_w // EB
    csz = NPAD // 16
    nss = nh * NPAD
    ssz = nss // 16
    zacc = jnp.zeros((NPAD, cw), jnp.float32)
    zss = jnp.zeros((nss,), jnp.float32)
    mesh = plsc.VectorSubcoreMesh(core_axis_name="c", subcore_axis_name="s")

    @functools.partial(
        pl.kernel, mesh=mesh,
        compiler_params=_SC_PARAMS,
        out_type=[jax.ShapeDtypeStruct((nc * 2 * NPAD, cw), jnp.float32),
                  jax.ShapeDtypeStruct((2 * nss,), jnp.float32)],
        scratch_types=[pltpu.VMEM((EB,), jnp.int32),
                       pltpu.VMEM((EB,), jnp.int32),
                       pltpu.VMEM((EB,), jnp.int32),
                       pltpu.VMEM((EB,), jnp.int32),
                       pltpu.VMEM((EB,), jnp.float32),
                       pltpu.VMEM((EB,), jnp.float32),
                       pltpu.VMEM((EB, cw), jnp.float32),
                       pltpu.VMEM((EB,), jnp.float32),
                       pltpu.VMEM((EB,), jnp.int32),
                       pltpu.VMEM_SHARED((NPAD, cw), jnp.float32),
                       pltpu.VMEM_SHARED((nss,), jnp.float32),
                       pltpu.SemaphoreType.DMA],
    )
    def k(tab_h, esed_h, src_h, dst_h, zacc_h, zss_h, acc_h, ssum_h,
          srcv, dstv, idxe, idxd, esb, edb, rows, pbuf, idxb,
          sh_acc, sh_ssum, sem):
        c = lax.axis_index("c")
        s = lax.axis_index("s")
        tilebase = c * (EPAD // 2) + s * per_w
        sc_off = pl.multiple_of(s * csz, 8)
        ss_off = pl.multiple_of(s * ssz, 8)
        pltpu.sync_copy(zss_h.at[pl.ds(ss_off, ssz)],
                        sh_ssum.at[pl.ds(ss_off, ssz)])

        def one_pass(ki, carry):
            hk = ki // (nc // nh)
            do_ssum = (ki % (nc // nh)) == 0
            tabk = tab_h.at[ki]
            pltpu.sync_copy(zacc_h.at[pl.ds(sc_off, csz)],
                            sh_acc.at[pl.ds(sc_off, csz)])
            plsc.subcore_barrier()

            def blk(b, cc):
                eoff = pl.multiple_of(tilebase + b * EB, 8)
                pltpu.sync_copy(src_h.at[pl.ds(eoff, EB)], srcv)
                pltpu.sync_copy(dst_h.at[pl.ds(eoff, EB)], dstv)
                for i in range(EB // 16):
                    sl = pl.ds(i * 16, 16)
                    idxe[sl] = srcv[sl] + hk * n
                    idxd[sl] = dstv[sl] + (nh + hk) * n
                pltpu.async_copy(esed_h.at[idxe], esb, sem).wait()
                pltpu.async_copy(esed_h.at[idxd], edb, sem).wait()
                pltpu.async_copy(tabk.at[srcv], rows, sem).wait()

                def slice_body(i, cc2):
                    sl = pl.ds(i * 16, 16)
                    dsts = dstv[sl]
                    q = esb[sl] + edb[sl]
                    pe16 = jnp.exp(jnp.maximum(q, 0.2 * q))

                    @pl.when(do_ssum)
                    def _():
                        pbuf[sl] = pe16
                        idxb[sl] = dsts * nh + hk

                    for e16 in range(16):
                        pe = pe16[e16]
                        r = i * 16 + e16
                        for j in range(cw // 16):
                            cs = pl.ds(j * 16, 16)
                            rows[r, cs] = rows[r, cs] * pe
                    return cc2

                lax.fori_loop(0, EB // 16, slice_body, 0)

                @pl.when(do_ssum)
                def _():
                    pltpu.sync_copy(pbuf, sh_ssum.at[idxb], add=True)

                pltpu.sync_copy(rows, sh_acc.at[dstv], add=True)
                return cc

            nreal = jnp.clip((e_real - tilebase) // EB, 0, nblk)
            lax.fori_loop(0, nreal, blk, 0)
            plsc.subcore_barrier()
            acc_off = pl.multiple_of((ki * 2 + c) * NPAD + s * csz, 8)
            pltpu.sync_copy(sh_acc.at[pl.ds(sc_off, csz)],
                            acc_h.at[pl.ds(acc_off, csz)])
            plsc.subcore_barrier()
            return carry

        lax.fori_loop(0, nc, one_pass, 0)
        pltpu.sync_copy(
            sh_ssum.at[pl.ds(ss_off, ssz)],
            ssum_h.at[pl.ds(pl.multiple_of(c * nss + s * ssz, 8), ssz)])

    return k(table, esed, src2, dst2, zacc, zss)


def _pair_gather(z, row, col):
    eq = row.shape[0]
    d = z.shape[1]
    per_w = eq // NW
    blk = 128
    mesh = plsc.VectorSubcoreMesh(core_axis_name="c", subcore_axis_name="s")

    @functools.partial(
        pl.kernel, mesh=mesh,
        out_type=[jax.ShapeDtypeStruct((eq, d), jnp.float32),
                  jax.ShapeDtypeStruct((eq, d), jnp.float32)],
        scratch_types=[pltpu.VMEM((blk,), jnp.int32),
                       pltpu.VMEM((blk, d), jnp.float32),
                       pltpu.SemaphoreType.DMA],
    )
    def k(z_h, row_h, col_h, zr_h, zc_h, idxv, rows, sem):
        wid = lax.axis_index("s") * 2 + lax.axis_index("c")
        base = wid * per_w

        def body(i, carry):
            off = base + i * blk
            pltpu.sync_copy(row_h.at[pl.ds(off, blk)], idxv)
            pltpu.async_copy(z_h.at[idxv], rows, sem).wait()
            pltpu.sync_copy(rows, zr_h.at[pl.ds(off, blk)])
            pltpu.sync_copy(col_h.at[pl.ds(off, blk)], idxv)
            pltpu.async_copy(z_h.at[idxv], rows, sem).wait()
            pltpu.sync_copy(rows, zc_h.at[pl.ds(off, blk)])
            return carry

        lax.fori_loop(0, per_w // blk, body, 0)

    return k(z, row, col)


def kernel(x, edge_index, edge_label_index, W1, a1s, a1d, b1, W2, a2s, a2d,
           b2, Wp1, bp1, g1, be1, Wp2, bp2, g2, be2, Wp3, bp3):
    n = x.shape[0]
    e = edge_index.shape[1]
    src = edge_index[0].astype(jnp.int32)
    dst = edge_index[1].astype(jnp.int32)
    src2 = jnp.pad(src, (0, EPAD - e))
    dst2 = jnp.pad(dst, (0, EPAD - e))

    h1 = _matmul(x, W1)
    h1h = h1.reshape(n, HEADS, HID)
    es1 = jnp.sum(h1h * a1s[None], axis=-1)
    ed1 = jnp.sum(h1h * a1d[None], axis=-1)
    esed1 = jnp.concatenate([es1.T.ravel(), ed1.T.ravel()])
    table1 = h1.reshape(n, 8, 128).transpose(1, 0, 2)
    acc1, ssum1p = _gat_layer(table1, esed1, src2, dst2, 8, HEADS, e)
    ssum1 = ssum1p.reshape(2, -1).sum(0)[:n * HEADS].reshape(n, HEADS)
    acc1 = acc1.reshape(8, 2, NPAD, 128).sum(1)[:, :n]
    out1 = acc1.transpose(1, 0, 2).reshape(n, HEADS, HID)
    out1 = out1 / (ssum1[:, :, None] + 1e-16)
    z = jax.nn.elu(out1.reshape(n, HEADS * HID) + b1)

    h2 = _matmul(z, W2)
    es2 = h2 @ a2s[0]
    ed2 = h2 @ a2d[0]
    esed2 = jnp.concatenate([es2, ed2])
    acc2, ssum2p = _gat_layer(h2[None], esed2, src2, dst2, 1, 1, e)
    ssum2 = ssum2p.reshape(2, -1).sum(0)[:n]
    acc2 = acc2.reshape(1, 2, NPAD, 128).sum(1)[0, :n]
    z2 = acc2 / (ssum2[:, None] + 1e-16) + b2

    row = edge_label_index[0].astype(jnp.int32)
    col = edge_label_index[1].astype(jnp.int32)
    zr, zc = _pair_gather(z2, row, col)
    ef = jnp.concatenate([zr, zc], axis=-1)
    bn_scale = 1.0 / jnp.sqrt(1.0 + 1e-5)
    h = jax.nn.relu((ef @ Wp1 + bp1) * bn_scale * g1 + be1)
    h = jax.nn.relu((h @ Wp2 + bp2) * bn_scale * g2 + be2)
    out = (h @ Wp3 + bp3).squeeze(-1)
    return out


# speedup vs baseline: 24.7415x; 2.0703x over previous
import functools

import jax
import jax.numpy as jnp
from jax import lax
from jax.experimental import pallas as pl
from jax.experimental.pallas import tpu as pltpu
from jax.experimental.pallas import tpu_sc as plsc

N = 10000
F_IN = 128
HID = 256
OUT = 128
HEADS = 4
NPAD = 10240
NW = 32
EB = 128
EPAD = 327680

_SC_PARAMS = pltpu.CompilerParams(needs_layout_passes=False)


def _mm_kernel(x_ref, w_ref, o_ref):
    o_ref[...] = jnp.dot(x_ref[...], w_ref[...],
                         preferred_element_type=jnp.float32)


def _matmul(x, w, bm=1000):
    m, k = x.shape
    _, n = w.shape
    return pl.pallas_call(
        _mm_kernel,
        grid=(m // bm,),
        in_specs=[pl.BlockSpec((bm, k), lambda i: (i, 0)),
                  pl.BlockSpec((k, n), lambda i: (0, 0))],
        out_specs=pl.BlockSpec((bm, n), lambda i: (i, 0)),
        out_shape=jax.ShapeDtypeStruct((m, n), jnp.float32),
    )(x, w)


def _gat_layer(table, esed, src2, dst2, nc, nh, e_real):
    n = table.shape[1]
    cw = table.shape[2]
    per_w = EPAD // NW
    nblk = per_w // EB
    csz = NPAD // 16
    nss = nh * NPAD
    ssz = nss // 16
    zacc = jnp.zeros((NPAD, cw), jnp.float32)
    zss = jnp.zeros((nss,), jnp.float32)
    mesh = plsc.VectorSubcoreMesh(core_axis_name="c", subcore_axis_name="s")

    @functools.partial(
        pl.kernel, mesh=mesh,
        compiler_params=_SC_PARAMS,
        out_type=[jax.ShapeDtypeStruct((nc * 2 * NPAD, cw), jnp.float32),
                  jax.ShapeDtypeStruct((2 * nss,), jnp.float32)],
        scratch_types=[pltpu.VMEM((2, EB), jnp.int32),
                       pltpu.VMEM((2, EB), jnp.int32),
                       pltpu.VMEM((2, EB), jnp.int32),
                       pltpu.VMEM((2, EB), jnp.int32),
                       pltpu.VMEM((2, EB), jnp.float32),
                       pltpu.VMEM((2, EB), jnp.float32),
                       pltpu.VMEM((EB, cw), jnp.float32),
                       pltpu.VMEM((EB, cw), jnp.float32),
                       pltpu.VMEM((EB,), jnp.float32),
                       pltpu.VMEM((EB,), jnp.int32),
                       pltpu.VMEM_SHARED((NPAD, cw), jnp.float32),
                       pltpu.VMEM_SHARED((nss,), jnp.float32),
                       pltpu.SemaphoreType.DMA,
                       pltpu.SemaphoreType.DMA,
                       pltpu.SemaphoreType.DMA,
                       pltpu.SemaphoreType.DMA,
                       pltpu.SemaphoreType.DMA,
                       pltpu.SemaphoreType.DMA],
    )
    def k(tab_h, esed_h, src_h, dst_h, zacc_h, zss_h, acc_h, ssum_h,
          srcv, dstv, idxe, idxd, esb, edb, rows0, rows1, pbuf, idxb,
          sh_acc, sh_ssum, sd0, sd1, ed0, ed1, r0, r1):
        c = lax.axis_index("c")
        s = lax.axis_index("s")
        tilebase = c * (EPAD // 2) + s * per_w
        sc_off = pl.multiple_of(s * csz, 8)
        ss_off = pl.multiple_of(s * ssz, 8)
        pltpu.sync_copy(zss_h.at[pl.ds(ss_off, ssz)],
                        sh_ssum.at[pl.ds(ss_off, ssz)])
        sd_sem = (sd0, sd1)
        ed_sem = (ed0, ed1)
        r_sem = (r0, r1)
        rows_b = (rows0, rows1)

        def inr(b):
            return jnp.logical_and(b < nblk, tilebase + b * EB < e_real)

        def one_pass(ki, carry):
            hk = ki // (nc // nh)
            do_ssum = (ki % (nc // nh)) == 0
            tabk = tab_h.at[ki]
            pltpu.sync_copy(zacc_h.at[pl.ds(sc_off, csz)],
                            sh_acc.at[pl.ds(sc_off, csz)])
            plsc.subcore_barrier()

            def start_sd(b, ph):
                @pl.when(inr(b))
                def _():
                    eoff = pl.multiple_of(tilebase + b * EB, 8)
                    pltpu.async_copy(src_h.at[pl.ds(eoff, EB)],
                                     srcv.at[ph], sd_sem[ph])
                    pltpu.async_copy(dst_h.at[pl.ds(eoff, EB)],
                                     dstv.at[ph], sd_sem[ph])

            def start_ged(b, ph):
                @pl.when(inr(b))
                def _():
                    eoff = pl.multiple_of(tilebase + b * EB, 8)
                    pltpu.make_async_copy(src_h.at[pl.ds(eoff, EB)],
                                          srcv.at[ph], sd_sem[ph]).wait()
                    pltpu.make_async_copy(dst_h.at[pl.ds(eoff, EB)],
                                          dstv.at[ph], sd_sem[ph]).wait()
                    for i in range(EB // 16):
                        sl = pl.ds(i * 16, 16)
                        idxe[ph, sl] = srcv[ph, sl] + hk * n
                        idxd[ph, sl] = dstv[ph, sl] + (nh + hk) * n
                    pltpu.async_copy(esed_h.at[idxe.at[ph]],
                                     esb.at[ph], ed_sem[ph])
                    pltpu.async_copy(esed_h.at[idxd.at[ph]],
                                     edb.at[ph], ed_sem[ph])
                    pltpu.async_copy(tabk.at[srcv.at[ph]],
                                     rows_b[ph], r_sem[ph])

            def proc(b, ph):
                @pl.when(inr(b))
                def _():
                    rows = rows_b[ph]
                    pltpu.make_async_copy(esed_h.at[idxe.at[ph]],
                                          esb.at[ph], ed_sem[ph]).wait()
                    pltpu.make_async_copy(esed_h.at[idxd.at[ph]],
                                          edb.at[ph], ed_sem[ph]).wait()
                    pltpu.make_async_copy(tabk.at[srcv.at[ph]],
                                          rows, r_sem[ph]).wait()

                    def slice_body(i, cc2):
                        sl = pl.ds(i * 16, 16)
                        dsts = dstv[ph, sl]
                        q = esb[ph, sl] + edb[ph, sl]
                        pe16 = jnp.exp(jnp.maximum(q, 0.2 * q))

                        @pl.when(do_ssum)
                        def _():
                            pbuf[sl] = pe16
                            idxb[sl] = dsts * nh + hk

                        for e16 in range(16):
                            pe = pe16[e16]
                            r = i * 16 + e16
                            for j in range(cw // 16):
                                cs = pl.ds(j * 16, 16)
                                rows[r, cs] = rows[r, cs] * pe
                        return cc2

                    lax.fori_loop(0, EB // 16, slice_body, 0)

                    @pl.when(do_ssum)
                    def _():
                        pltpu.sync_copy(pbuf, sh_ssum.at[idxb], add=True)

                    pltpu.sync_copy(rows, sh_acc.at[dstv.at[ph]], add=True)

            start_sd(0, 0)
            start_ged(0, 0)
            start_sd(1, 1)

            def pair(ib2, cc):
                b0 = ib2 * 2
                b1 = b0 + 1
                start_ged(b1, 1)
                proc(b0, 0)
                start_sd(b0 + 2, 0)
                start_ged(b0 + 2, 0)
                proc(b1, 1)
                start_sd(b1 + 2, 1)
                return cc

            lax.fori_loop(0, nblk // 2, pair, 0)
            plsc.subcore_barrier()
            acc_off = pl.multiple_of((ki * 2 + c) * NPAD + s * csz, 8)
            pltpu.sync_copy(sh_acc.at[pl.ds(sc_off, csz)],
                            acc_h.at[pl.ds(acc_off, csz)])
            plsc.subcore_barrier()
            return carry

        lax.fori_loop(0, nc, one_pass, 0)
        pltpu.sync_copy(
            sh_ssum.at[pl.ds(ss_off, ssz)],
            ssum_h.at[pl.ds(pl.multiple_of(c * nss + s * ssz, 8), ssz)])

    return k(table, esed, src2, dst2, zacc, zss)


def _pair_gather(z, row, col):
    eq = row.shape[0]
    d = z.shape[1]
    per_w = eq // NW
    blk = 128
    mesh = plsc.VectorSubcoreMesh(core_axis_name="c", subcore_axis_name="s")

    @functools.partial(
        pl.kernel, mesh=mesh,
        out_type=[jax.ShapeDtypeStruct((eq, d), jnp.float32),
                  jax.ShapeDtypeStruct((eq, d), jnp.float32)],
        scratch_types=[pltpu.VMEM((blk,), jnp.int32),
                       pltpu.VMEM((blk, d), jnp.float32),
                       pltpu.SemaphoreType.DMA],
    )
    def k(z_h, row_h, col_h, zr_h, zc_h, idxv, rows, sem):
        wid = lax.axis_index("s") * 2 + lax.axis_index("c")
        base = wid * per_w

        def body(i, carry):
            off = base + i * blk
            pltpu.sync_copy(row_h.at[pl.ds(off, blk)], idxv)
            pltpu.async_copy(z_h.at[idxv], rows, sem).wait()
            pltpu.sync_copy(rows, zr_h.at[pl.ds(off, blk)])
            pltpu.sync_copy(col_h.at[pl.ds(off, blk)], idxv)
            pltpu.async_copy(z_h.at[idxv], rows, sem).wait()
            pltpu.sync_copy(rows, zc_h.at[pl.ds(off, blk)])
            return carry

        lax.fori_loop(0, per_w // blk, body, 0)

    return k(z, row, col)


def kernel(x, edge_index, edge_label_index, W1, a1s, a1d, b1, W2, a2s, a2d,
           b2, Wp1, bp1, g1, be1, Wp2, bp2, g2, be2, Wp3, bp3):
    n = x.shape[0]
    e = edge_index.shape[1]
    src = edge_index[0].astype(jnp.int32)
    dst = edge_index[1].astype(jnp.int32)
    src2 = jnp.pad(src, (0, EPAD - e))
    dst2 = jnp.pad(dst, (0, EPAD - e))

    h1 = _matmul(x, W1)
    h1h = h1.reshape(n, HEADS, HID)
    es1 = jnp.sum(h1h * a1s[None], axis=-1)
    ed1 = jnp.sum(h1h * a1d[None], axis=-1)
    esed1 = jnp.concatenate([es1.T.ravel(), ed1.T.ravel()])
    table1 = h1.reshape(n, 8, 128).transpose(1, 0, 2)
    acc1, ssum1p = _gat_layer(table1, esed1, src2, dst2, 8, HEADS, e)
    ssum1 = ssum1p.reshape(2, -1).sum(0)[:n * HEADS].reshape(n, HEADS)
    acc1 = acc1.reshape(8, 2, NPAD, 128).sum(1)[:, :n]
    out1 = acc1.transpose(1, 0, 2).reshape(n, HEADS, HID)
    out1 = out1 / (ssum1[:, :, None] + 1e-16)
    z = jax.nn.elu(out1.reshape(n, HEADS * HID) + b1)

    h2 = _matmul(z, W2)
    es2 = h2 @ a2s[0]
    ed2 = h2 @ a2d[0]
    esed2 = jnp.concatenate([es2, ed2])
    acc2, ssum2p = _gat_layer(h2[None], esed2, src2, dst2, 1, 1, e)
    ssum2 = ssum2p.reshape(2, -1).sum(0)[:n]
    acc2 = acc2.reshape(1, 2, NPAD, 128).sum(1)[0, :n]
    z2 = acc2 / (ssum2[:, None] + 1e-16) + b2

    row = edge_label_index[0].astype(jnp.int32)
    col = edge_label_index[1].astype(jnp.int32)
    zr, zc = _pair_gather(z2, row, col)
    ef = jnp.concatenate([zr, zc], axis=-1)
    bn_scale = 1.0 / jnp.sqrt(1.0 + 1e-5)
    h = jax.nn.relu((ef @ Wp1 + bp1) * bn_scale * g1 + be1)
    h = jax.nn.relu((h @ Wp2 + bp2) * bn_scale * g2 + be2)
    out = (h @ Wp3 + bp3).squeeze(-1)
    return out


# speedup vs baseline: 25.6997x; 1.0387x over previous
import functools

import jax
import jax.numpy as jnp
from jax import lax
from jax.experimental import pallas as pl
from jax.experimental.pallas import tpu as pltpu
from jax.experimental.pallas import tpu_sc as plsc

N = 10000
F_IN = 128
HID = 256
OUT = 128
HEADS = 4
NPAD = 10240
NW = 32
EB = 128
EPAD = 327680

_SC_PARAMS = pltpu.CompilerParams(needs_layout_passes=False)


def _mm_kernel(x_ref, w_ref, o_ref):
    o_ref[...] = jnp.dot(x_ref[...], w_ref[...],
                         preferred_element_type=jnp.float32)


def _matmul(x, w, bm=1000):
    m, k = x.shape
    _, n = w.shape
    return pl.pallas_call(
        _mm_kernel,
        grid=(m // bm,),
        in_specs=[pl.BlockSpec((bm, k), lambda i: (i, 0)),
                  pl.BlockSpec((k, n), lambda i: (0, 0))],
        out_specs=pl.BlockSpec((bm, n), lambda i: (i, 0)),
        out_shape=jax.ShapeDtypeStruct((m, n), jnp.float32),
    )(x, w)


def _enc_kernel(x_ref, w_ref, as_ref, ad_ref, h_ref, es_ref, ed_ref):
    h = jnp.dot(x_ref[...], w_ref[0],
                preferred_element_type=jnp.float32)
    h_ref[0] = h
    es_ref[...] = (h @ as_ref[0, 0])[None, :, None]
    ed_ref[...] = (h @ ad_ref[0, 0])[None, :, None]


def _encode(x, w8, as8, ad8, bm=1000):
    m, f = x.shape
    nc = w8.shape[0]
    nb = m // bm
    h8, es, ed = pl.pallas_call(
        _enc_kernel,
        grid=(nb, nc),
        in_specs=[pl.BlockSpec((bm, f), lambda i, j: (i, 0)),
                  pl.BlockSpec((1, f, 128), lambda i, j: (j, 0, 0)),
                  pl.BlockSpec((1, 1, 128), lambda i, j: (j, 0, 0)),
                  pl.BlockSpec((1, 1, 128), lambda i, j: (j, 0, 0))],
        out_specs=[pl.BlockSpec((1, bm, 128), lambda i, j: (j, i, 0)),
                   pl.BlockSpec((1, bm, 1), lambda i, j: (j, i, 0)),
                   pl.BlockSpec((1, bm, 1), lambda i, j: (j, i, 0))],
        out_shape=[jax.ShapeDtypeStruct((nc, m, 128), jnp.float32),
                   jax.ShapeDtypeStruct((nc, m, 1), jnp.float32),
                   jax.ShapeDtypeStruct((nc, m, 1), jnp.float32)],
    )(x, w8, as8[:, None], ad8[:, None])
    return h8, es.reshape(nc, m), ed.reshape(nc, m)


def _dec_kernel(zr_ref, zc_ref, wa_ref, wb_ref, b1_ref, g1_ref, e1_ref,
                w2_ref, b2_ref, g2_ref, e2_ref, w3_ref, b3_ref, o_ref):
    h = (jnp.dot(zr_ref[...], wa_ref[...],
                 preferred_element_type=jnp.float32)
         + jnp.dot(zc_ref[...], wb_ref[...],
                   preferred_element_type=jnp.float32) + b1_ref[0])
    h = jax.nn.relu(h * g1_ref[0] + e1_ref[0])
    h = jnp.dot(h, w2_ref[...], preferred_element_type=jnp.float32)
    h = jax.nn.relu((h + b2_ref[0]) * g2_ref[0] + e2_ref[0])
    o_ref[...] = (h @ w3_ref[...] + b3_ref[0])


def _decoder(zr, zc, wa, wb, b1, g1s, e1, w2, b2, g2s, e2, w3, b3, bq=2048):
    eq = zr.shape[0]
    full = lambda i: (0, 0)
    return pl.pallas_call(
        _dec_kernel,
        grid=(eq // bq,),
        in_specs=[pl.BlockSpec((bq, 128), lambda i: (i, 0)),
                  pl.BlockSpec((bq, 128), lambda i: (i, 0)),
                  pl.BlockSpec((128, HID), full),
                  pl.BlockSpec((128, HID), full),
                  pl.BlockSpec((1, HID), full),
                  pl.BlockSpec((1, HID), full),
                  pl.BlockSpec((1, HID), full),
                  pl.BlockSpec((HID, 32), full),
                  pl.BlockSpec((1, 32), full),
                  pl.BlockSpec((1, 32), full),
                  pl.BlockSpec((1, 32), full),
                  pl.BlockSpec((32, 1), full),
                  pl.BlockSpec((1, 1), full)],
        out_specs=pl.BlockSpec((bq, 1), lambda i: (i, 0)),
        out_shape=jax.ShapeDtypeStruct((eq, 1), jnp.float32),
    )(zr, zc, wa, wb, b1[None], g1s[None], e1[None], w2, b2[None],
      g2s[None], e2[None], w3, b3[None])


def _gat_layer(table, esed, src2, dst2, nc, nh, e_real):
    n = table.shape[1]
    cw = table.shape[2]
    per_w = EPAD // NW
    nblk = per_w // EB
    csz = NPAD // 16
    nss = nh * NPAD
    ssz = nss // 16
    zacc = jnp.zeros((NPAD, cw), jnp.float32)
    zss = jnp.zeros((nss,), jnp.float32)
    mesh = plsc.VectorSubcoreMesh(core_axis_name="c", subcore_axis_name="s")

    @functools.partial(
        pl.kernel, mesh=mesh,
        compiler_params=_SC_PARAMS,
        out_type=[jax.ShapeDtypeStruct((nc * 2 * NPAD, cw), jnp.float32),
                  jax.ShapeDtypeStruct((2 * nss,), jnp.float32)],
        scratch_types=[pltpu.VMEM((2, EB), jnp.int32),
                       pltpu.VMEM((2, EB), jnp.int32),
                       pltpu.VMEM((2, EB), jnp.int32),
                       pltpu.VMEM((2, EB), jnp.int32),
                       pltpu.VMEM((2, EB), jnp.float32),
                       pltpu.VMEM((2, EB), jnp.float32),
                       pltpu.VMEM((EB, cw), jnp.float32),
                       pltpu.VMEM((EB, cw), jnp.float32),
                       pltpu.VMEM((EB,), jnp.float32),
                       pltpu.VMEM((EB,), jnp.int32),
                       pltpu.VMEM_SHARED((NPAD, cw), jnp.float32),
                       pltpu.VMEM_SHARED((nss,), jnp.float32),
                       pltpu.SemaphoreType.DMA,
                       pltpu.SemaphoreType.DMA,
                       pltpu.SemaphoreType.DMA,
                       pltpu.SemaphoreType.DMA,
                       pltpu.SemaphoreType.DMA,
                       pltpu.SemaphoreType.DMA],
    )
    def k(tab_h, esed_h, src_h, dst_h, zacc_h, zss_h, acc_h, ssum_h,
          srcv, dstv, idxe, idxd, esb, edb, rows0, rows1, pbuf, idxb,
          sh_acc, sh_ssum, sd0, sd1, ed0, ed1, r0, r1):
        c = lax.axis_index("c")
        s = lax.axis_index("s")
        tilebase = c * (EPAD // 2) + s * per_w
        sc_off = pl.multiple_of(s * csz, 8)
        ss_off = pl.multiple_of(s * ssz, 8)
        pltpu.sync_copy(zss_h.at[pl.ds(ss_off, ssz)],
                        sh_ssum.at[pl.ds(ss_off, ssz)])
        sd_sem = (sd0, sd1)
        ed_sem = (ed0, ed1)
        r_sem = (r0, r1)
        rows_b = (rows0, rows1)

        def inr(b):
            return jnp.logical_and(b < nblk, tilebase + b * EB < e_real)

        def one_pass(ki, carry):
            hk = ki // (nc // nh)
            do_ssum = (ki % (nc // nh)) == 0
            tabk = tab_h.at[ki]
            pltpu.sync_copy(zacc_h.at[pl.ds(sc_off, csz)],
                            sh_acc.at[pl.ds(sc_off, csz)])
            plsc.subcore_barrier()

            def start_sd(b, ph):
                @pl.when(inr(b))
                def _():
                    eoff = pl.multiple_of(tilebase + b * EB, 8)
                    pltpu.async_copy(src_h.at[pl.ds(eoff, EB)],
                                     srcv.at[ph], sd_sem[ph])
                    pltpu.async_copy(dst_h.at[pl.ds(eoff, EB)],
                                     dstv.at[ph], sd_sem[ph])

            def start_ged(b, ph):
                @pl.when(inr(b))
                def _():
                    eoff = pl.multiple_of(tilebase + b * EB, 8)
                    pltpu.make_async_copy(src_h.at[pl.ds(eoff, EB)],
                                          srcv.at[ph], sd_sem[ph]).wait()
                    pltpu.make_async_copy(dst_h.at[pl.ds(eoff, EB)],
                                          dstv.at[ph], sd_sem[ph]).wait()
                    for i in range(EB // 16):
                        sl = pl.ds(i * 16, 16)
                        idxe[ph, sl] = srcv[ph, sl] + hk * n
                        idxd[ph, sl] = dstv[ph, sl] + (nh + hk) * n
                    pltpu.async_copy(esed_h.at[idxe.at[ph]],
                                     esb.at[ph], ed_sem[ph])
                    pltpu.async_copy(esed_h.at[idxd.at[ph]],
                                     edb.at[ph], ed_sem[ph])
                    pltpu.async_copy(tabk.at[srcv.at[ph]],
                                     rows_b[ph], r_sem[ph])

            def proc(b, ph):
                @pl.when(inr(b))
                def _():
                    rows = rows_b[ph]
                    pltpu.make_async_copy(esed_h.at[idxe.at[ph]],
                                          esb.at[ph], ed_sem[ph]).wait()
                    pltpu.make_async_copy(esed_h.at[idxd.at[ph]],
                                          edb.at[ph], ed_sem[ph]).wait()
                    pltpu.make_async_copy(tabk.at[srcv.at[ph]],
                                          rows, r_sem[ph]).wait()

                    def slice_body(i, cc2):
                        sl = pl.ds(i * 16, 16)
                        dsts = dstv[ph, sl]
                        q = esb[ph, sl] + edb[ph, sl]
                        pe16 = jnp.exp(jnp.maximum(q, 0.2 * q))

                        @pl.when(do_ssum)
                        def _():
                            pbuf[sl] = pe16
                            idxb[sl] = dsts * nh + hk

                        for e16 in range(16):
                            pe = pe16[e16]
                            r = i * 16 + e16
                            for j in range(cw // 16):
                                cs = pl.ds(j * 16, 16)
                                rows[r, cs] = rows[r, cs] * pe
                        return cc2

                    lax.fori_loop(0, EB // 16, slice_body, 0)

                    @pl.when(do_ssum)
                    def _():
                        pltpu.sync_copy(pbuf, sh_ssum.at[idxb], add=True)

                    pltpu.sync_copy(rows, sh_acc.at[dstv.at[ph]], add=True)

            start_sd(0, 0)
            start_ged(0, 0)
            start_sd(1, 1)

            def pair(ib2, cc):
                b0 = ib2 * 2
                b1 = b0 + 1
                start_ged(b1, 1)
                proc(b0, 0)
                start_sd(b0 + 2, 0)
                start_ged(b0 + 2, 0)
                proc(b1, 1)
                start_sd(b1 + 2, 1)
                return cc

            lax.fori_loop(0, nblk // 2, pair, 0)
            plsc.subcore_barrier()
            acc_off = pl.multiple_of((ki * 2 + c) * NPAD + s * csz, 8)
            pltpu.sync_copy(sh_acc.at[pl.ds(sc_off, csz)],
                            acc_h.at[pl.ds(acc_off, csz)])
            plsc.subcore_barrier()
            return carry

        lax.fori_loop(0, nc, one_pass, 0)
        pltpu.sync_copy(
            sh_ssum.at[pl.ds(ss_off, ssz)],
            ssum_h.at[pl.ds(pl.multiple_of(c * nss + s * ssz, 8), ssz)])

    return k(table, esed, src2, dst2, zacc, zss)


def _pair_gather(z, row, col):
    eq = row.shape[0]
    d = z.shape[1]
    per_w = eq // NW
    blk = 128
    mesh = plsc.VectorSubcoreMesh(core_axis_name="c", subcore_axis_name="s")

    @functools.partial(
        pl.kernel, mesh=mesh,
        out_type=[jax.ShapeDtypeStruct((eq, d), jnp.float32),
                  jax.ShapeDtypeStruct((eq, d), jnp.float32)],
        scratch_types=[pltpu.VMEM((blk,), jnp.int32),
                       pltpu.VMEM((blk, d), jnp.float32),
                       pltpu.SemaphoreType.DMA],
    )
    def k(z_h, row_h, col_h, zr_h, zc_h, idxv, rows, sem):
        wid = lax.axis_index("s") * 2 + lax.axis_index("c")
        base = wid * per_w

        def body(i, carry):
            off = base + i * blk
            pltpu.sync_copy(row_h.at[pl.ds(off, blk)], idxv)
            pltpu.async_copy(z_h.at[idxv], rows, sem).wait()
            pltpu.sync_copy(rows, zr_h.at[pl.ds(off, blk)])
            pltpu.sync_copy(col_h.at[pl.ds(off, blk)], idxv)
            pltpu.async_copy(z_h.at[idxv], rows, sem).wait()
            pltpu.sync_copy(rows, zc_h.at[pl.ds(off, blk)])
            return carry

        lax.fori_loop(0, per_w // blk, body, 0)

    return k(z, row, col)


def kernel(x, edge_index, edge_label_index, W1, a1s, a1d, b1, W2, a2s, a2d,
           b2, Wp1, bp1, g1, be1, Wp2, bp2, g2, be2, Wp3, bp3):
    n = x.shape[0]
    e = edge_index.shape[1]
    src = edge_index[0].astype(jnp.int32)
    dst = edge_index[1].astype(jnp.int32)
    src2 = jnp.pad(src, (0, EPAD - e))
    dst2 = jnp.pad(dst, (0, EPAD - e))

    w18 = W1.reshape(F_IN, 8, 128).transpose(1, 0, 2)
    a1s8 = a1s.reshape(8, 128)
    a1d8 = a1d.reshape(8, 128)
    table1, esp, edp = _encode(x, w18, a1s8, a1d8)
    esed1 = jnp.concatenate([esp.reshape(4, 2, n).sum(1).ravel(),
                             edp.reshape(4, 2, n).sum(1).ravel()])
    acc1, ssum1p = _gat_layer(table1, esed1, src2, dst2, 8, HEADS, e)
    ssum1 = ssum1p.reshape(2, -1).sum(0)[:n * HEADS].reshape(n, HEADS)
    acc1 = acc1.reshape(8, 2, NPAD, 128).sum(1)[:, :n]
    out1 = acc1.transpose(1, 0, 2).reshape(n, HEADS, HID)
    out1 = out1 / (ssum1[:, :, None] + 1e-16)
    z = jax.nn.elu(out1.reshape(n, HEADS * HID) + b1)

    h2 = _matmul(z, W2)
    es2 = h2 @ a2s[0]
    ed2 = h2 @ a2d[0]
    esed2 = jnp.concatenate([es2, ed2])
    acc2, ssum2p = _gat_layer(h2[None], esed2, src2, dst2, 1, 1, e)
    ssum2 = ssum2p.reshape(2, -1).sum(0)[:n]
    acc2 = acc2.reshape(1, 2, NPAD, 128).sum(1)[0, :n]
    z2 = acc2 / (ssum2[:, None] + 1e-16) + b2

    row = edge_label_index[0].astype(jnp.int32)
    col = edge_label_index[1].astype(jnp.int32)
    zr, zc = _pair_gather(z2, row, col)
    bn_scale = 1.0 / jnp.sqrt(1.0 + 1e-5)
    out = _decoder(zr, zc, Wp1[:128], Wp1[128:], bp1, bn_scale * g1, be1,
                   Wp2, bp2, bn_scale * g2, be2, Wp3, bp3)
    return out[:, 0]


# speedup vs baseline: 30.1240x; 1.1722x over previous
import functools

import jax
import jax.numpy as jnp
from jax import lax
from jax.experimental import pallas as pl
from jax.experimental.pallas import tpu as pltpu
from jax.experimental.pallas import tpu_sc as plsc

N = 10000
F_IN = 128
HID = 256
OUT = 128
HEADS = 4
NPAD = 10240
NW = 32
EB = 128
EPAD = 327680

_SC_PARAMS = pltpu.CompilerParams(needs_layout_passes=False)


def _mm_kernel(x_ref, w_ref, o_ref):
    o_ref[...] = jnp.dot(x_ref[...], w_ref[...],
                         preferred_element_type=jnp.float32)


def _matmul(x, w, bm=1000):
    m, k = x.shape
    _, n = w.shape
    return pl.pallas_call(
        _mm_kernel,
        grid=(m // bm,),
        in_specs=[pl.BlockSpec((bm, k), lambda i: (i, 0)),
                  pl.BlockSpec((k, n), lambda i: (0, 0))],
        out_specs=pl.BlockSpec((bm, n), lambda i: (i, 0)),
        out_shape=jax.ShapeDtypeStruct((m, n), jnp.float32),
    )(x, w)


def _enc_kernel(x_ref, w_ref, as_ref, ad_ref, h_ref, es_ref, ed_ref):
    h = jnp.dot(x_ref[...], w_ref[0],
                preferred_element_type=jnp.float32)
    h_ref[0] = h
    es_ref[...] = (h @ as_ref[0, 0])[None, :, None]
    ed_ref[...] = (h @ ad_ref[0, 0])[None, :, None]


def _encode(x, w8, as8, ad8, bm=1000):
    m, f = x.shape
    nc = w8.shape[0]
    nb = m // bm
    h8, es, ed = pl.pallas_call(
        _enc_kernel,
        grid=(nb, nc),
        in_specs=[pl.BlockSpec((bm, f), lambda i, j: (i, 0)),
                  pl.BlockSpec((1, f, 128), lambda i, j: (j, 0, 0)),
                  pl.BlockSpec((1, 1, 128), lambda i, j: (j, 0, 0)),
                  pl.BlockSpec((1, 1, 128), lambda i, j: (j, 0, 0))],
        out_specs=[pl.BlockSpec((1, bm, 128), lambda i, j: (j, i, 0)),
                   pl.BlockSpec((1, bm, 1), lambda i, j: (j, i, 0)),
                   pl.BlockSpec((1, bm, 1), lambda i, j: (j, i, 0))],
        out_shape=[jax.ShapeDtypeStruct((nc, m, 128), jnp.float32),
                   jax.ShapeDtypeStruct((nc, m, 1), jnp.float32),
                   jax.ShapeDtypeStruct((nc, m, 1), jnp.float32)],
    )(x, w8, as8[:, None], ad8[:, None])
    return h8, es.reshape(nc, m), ed.reshape(nc, m)


def _dec_kernel(zr_ref, zc_ref, wa_ref, wb_ref, b1_ref, g1_ref, e1_ref,
                w2_ref, b2_ref, g2_ref, e2_ref, w3_ref, b3_ref, o_ref):
    h = (jnp.dot(zr_ref[...], wa_ref[...],
                 preferred_element_type=jnp.float32)
         + jnp.dot(zc_ref[...], wb_ref[...],
                   preferred_element_type=jnp.float32) + b1_ref[0])
    h = jax.nn.relu(h * g1_ref[0] + e1_ref[0])
    h = jnp.dot(h, w2_ref[...], preferred_element_type=jnp.float32)
    h = jax.nn.relu((h + b2_ref[0]) * g2_ref[0] + e2_ref[0])
    o_ref[...] = (h @ w3_ref[...] + b3_ref[0])


def _decoder(zr, zc, wa, wb, b1, g1s, e1, w2, b2, g2s, e2, w3, b3, bq=2048):
    eq = zr.shape[0]
    full = lambda i: (0, 0)
    return pl.pallas_call(
        _dec_kernel,
        grid=(eq // bq,),
        in_specs=[pl.BlockSpec((bq, 128), lambda i: (i, 0)),
                  pl.BlockSpec((bq, 128), lambda i: (i, 0)),
                  pl.BlockSpec((128, HID), full),
                  pl.BlockSpec((128, HID), full),
                  pl.BlockSpec((1, HID), full),
                  pl.BlockSpec((1, HID), full),
                  pl.BlockSpec((1, HID), full),
                  pl.BlockSpec((HID, 32), full),
                  pl.BlockSpec((1, 32), full),
                  pl.BlockSpec((1, 32), full),
                  pl.BlockSpec((1, 32), full),
                  pl.BlockSpec((32, 1), full),
                  pl.BlockSpec((1, 1), full)],
        out_specs=pl.BlockSpec((bq, 1), lambda i: (i, 0)),
        out_shape=jax.ShapeDtypeStruct((eq, 1), jnp.float32),
    )(zr, zc, wa, wb, b1[None], g1s[None], e1[None], w2, b2[None],
      g2s[None], e2[None], w3, b3[None])


def _gat_layer(table, esed, src2, dst2, nc, nh, e_real):
    n = table.shape[1]
    cw = table.shape[2]
    per_w = EPAD // NW
    nblk = per_w // EB
    csz = NPAD // 16
    nss = nh * NPAD
    ssz = nss // 16
    zacc = jnp.zeros((NPAD, cw), jnp.float32)
    zss = jnp.zeros((nss,), jnp.float32)
    mesh = plsc.VectorSubcoreMesh(core_axis_name="c", subcore_axis_name="s")

    @functools.partial(
        pl.kernel, mesh=mesh,
        compiler_params=_SC_PARAMS,
        out_type=[jax.ShapeDtypeStruct((nc * 2 * NPAD, cw), jnp.float32),
                  jax.ShapeDtypeStruct((2 * nss,), jnp.float32)],
        scratch_types=[pltpu.VMEM((2, EB), jnp.int32),
                       pltpu.VMEM((2, EB), jnp.int32),
                       pltpu.VMEM((2, EB), jnp.int32),
                       pltpu.VMEM((2, EB), jnp.int32),
                       pltpu.VMEM((2, EB), jnp.float32),
                       pltpu.VMEM((2, EB), jnp.float32),
                       pltpu.VMEM((EB, cw), jnp.float32),
                       pltpu.VMEM((EB, cw), jnp.float32),
                       pltpu.VMEM((EB,), jnp.float32),
                       pltpu.VMEM((EB,), jnp.int32),
                       pltpu.VMEM((EB,), jnp.int32),
                       pltpu.VMEM_SHARED((NPAD, cw), jnp.float32),
                       pltpu.VMEM_SHARED((nss,), jnp.float32),
                       pltpu.SemaphoreType.DMA,
                       pltpu.SemaphoreType.DMA,
                       pltpu.SemaphoreType.DMA,
                       pltpu.SemaphoreType.DMA,
                       pltpu.SemaphoreType.DMA,
                       pltpu.SemaphoreType.DMA],
    )
    def k(tab_h, esed_h, src_h, dst_h, zacc_h, zss_h, acc_h, ssum_h,
          srcv, dstv, idxe, idxd, esb, edb, rows0, rows1, pbuf, idxb,
          dvloc, sh_acc, sh_ssum, sd0, sd1, ed0, ed1, r0, r1):
        c = lax.axis_index("c")
        s = lax.axis_index("s")
        tilebase = c * (EPAD // 2) + s * per_w
        sc_off = pl.multiple_of(s * csz, 8)
        ss_off = pl.multiple_of(s * ssz, 8)
        pltpu.sync_copy(zss_h.at[pl.ds(ss_off, ssz)],
                        sh_ssum.at[pl.ds(ss_off, ssz)])
        sd_sem = (sd0, sd1)
        ed_sem = (ed0, ed1)
        r_sem = (r0, r1)
        rows_b = (rows0, rows1)

        def inr(b):
            return jnp.logical_and(b < nblk, tilebase + b * EB < e_real)

        def one_pass(ki, carry):
            hk = ki // (nc // nh)
            do_ssum = (ki % (nc // nh)) == 0
            tabk = tab_h.at[ki]
            pltpu.sync_copy(zacc_h.at[pl.ds(sc_off, csz)],
                            sh_acc.at[pl.ds(sc_off, csz)])
            plsc.subcore_barrier()

            def start_sd(b, ph):
                @pl.when(inr(b))
                def _():
                    eoff = pl.multiple_of(tilebase + b * EB, 8)
                    pltpu.async_copy(src_h.at[pl.ds(eoff, EB)],
                                     srcv.at[ph], sd_sem[ph])
                    pltpu.async_copy(dst_h.at[pl.ds(eoff, EB)],
                                     dstv.at[ph], sd_sem[ph])

            def start_ged(b, ph):
                @pl.when(inr(b))
                def _():
                    eoff = pl.multiple_of(tilebase + b * EB, 8)
                    pltpu.make_async_copy(src_h.at[pl.ds(eoff, EB)],
                                          srcv.at[ph], sd_sem[ph]).wait()
                    pltpu.make_async_copy(dst_h.at[pl.ds(eoff, EB)],
                                          dstv.at[ph], sd_sem[ph]).wait()
                    for i in range(EB // 16):
                        sl = pl.ds(i * 16, 16)
                        idxe[ph, sl] = srcv[ph, sl] + hk * n
                        idxd[ph, sl] = dstv[ph, sl] + (nh + hk) * n
                    pltpu.async_copy(esed_h.at[idxe.at[ph]],
                                     esb.at[ph], ed_sem[ph])
                    pltpu.async_copy(esed_h.at[idxd.at[ph]],
                                     edb.at[ph], ed_sem[ph])
                    pltpu.async_copy(tabk.at[srcv.at[ph]],
                                     rows_b[ph], r_sem[ph])

            def step(b, ph):
                rows = rows_b[ph]

                @pl.when(inr(b))
                def _():
                    pltpu.make_async_copy(esed_h.at[idxe.at[ph]],
                                          esb.at[ph], ed_sem[ph]).wait()
                    pltpu.make_async_copy(esed_h.at[idxd.at[ph]],
                                          edb.at[ph], ed_sem[ph]).wait()
                    pltpu.make_async_copy(tabk.at[srcv.at[ph]],
                                          rows, r_sem[ph]).wait()
                    for i in range(EB // 16):
                        sl = pl.ds(i * 16, 16)
                        dvloc[sl] = dstv[ph, sl]

                start_sd(b + 2, ph)

                @pl.when(inr(b))
                def _():
                    def slice_body(i, cc2):
                        sl = pl.ds(i * 16, 16)
                        dsts = dvloc[sl]
                        q = esb[ph, sl] + edb[ph, sl]
                        pe16 = jnp.exp(jnp.maximum(q, 0.2 * q))

                        @pl.when(do_ssum)
                        def _():
                            pbuf[sl] = pe16
                            idxb[sl] = dsts * nh + hk

                        for e16 in range(16):
                            pe = pe16[e16]
                            r = i * 16 + e16
                            for j in range(cw // 16):
                                cs = pl.ds(j * 16, 16)
                                rows[r, cs] = rows[r, cs] * pe
                        return cc2

                    lax.fori_loop(0, EB // 16, slice_body, 0)

                    @pl.when(do_ssum)
                    def _():
                        pltpu.sync_copy(pbuf, sh_ssum.at[idxb], add=True)

                    pltpu.sync_copy(rows, sh_acc.at[dvloc], add=True)

                start_ged(b + 2, ph)

            start_sd(0, 0)
            start_ged(0, 0)
            start_sd(1, 1)
            start_ged(1, 1)

            def pair(ib2, cc):
                b0 = ib2 * 2
                step(b0, 0)
                step(b0 + 1, 1)
                return cc

            lax.fori_loop(0, nblk // 2, pair, 0)
            plsc.subcore_barrier()
            acc_off = pl.multiple_of((ki * 2 + c) * NPAD + s * csz, 8)
            pltpu.sync_copy(sh_acc.at[pl.ds(sc_off, csz)],
                            acc_h.at[pl.ds(acc_off, csz)])
            plsc.subcore_barrier()
            return carry

        lax.fori_loop(0, nc, one_pass, 0)
        pltpu.sync_copy(
            sh_ssum.at[pl.ds(ss_off, ssz)],
            ssum_h.at[pl.ds(pl.multiple_of(c * nss + s * ssz, 8), ssz)])

    return k(table, esed, src2, dst2, zacc, zss)


def _pair_gather(z, row, col):
    eq = row.shape[0]
    d = z.shape[1]
    per_w = eq // NW
    blk = 128
    mesh = plsc.VectorSubcoreMesh(core_axis_name="c", subcore_axis_name="s")

    @functools.partial(
        pl.kernel, mesh=mesh,
        out_type=[jax.ShapeDtypeStruct((eq, d), jnp.float32),
                  jax.ShapeDtypeStruct((eq, d), jnp.float32)],
        scratch_types=[pltpu.VMEM((blk,), jnp.int32),
                       pltpu.VMEM((blk, d), jnp.float32),
                       pltpu.SemaphoreType.DMA],
    )
    def k(z_h, row_h, col_h, zr_h, zc_h, idxv, rows, sem):
        wid = lax.axis_index("s") * 2 + lax.axis_index("c")
        base = wid * per_w

        def body(i, carry):
            off = base + i * blk
            pltpu.sync_copy(row_h.at[pl.ds(off, blk)], idxv)
            pltpu.async_copy(z_h.at[idxv], rows, sem).wait()
            pltpu.sync_copy(rows, zr_h.at[pl.ds(off, blk)])
            pltpu.sync_copy(col_h.at[pl.ds(off, blk)], idxv)
            pltpu.async_copy(z_h.at[idxv], rows, sem).wait()
            pltpu.sync_copy(rows, zc_h.at[pl.ds(off, blk)])
            return carry

        lax.fori_loop(0, per_w // blk, body, 0)

    return k(z, row, col)


def kernel(x, edge_index, edge_label_index, W1, a1s, a1d, b1, W2, a2s, a2d,
           b2, Wp1, bp1, g1, be1, Wp2, bp2, g2, be2, Wp3, bp3):
    n = x.shape[0]
    e = edge_index.shape[1]
    src = edge_index[0].astype(jnp.int32)
    dst = edge_index[1].astype(jnp.int32)
    src2 = jnp.pad(src, (0, EPAD - e))
    dst2 = jnp.pad(dst, (0, EPAD - e))

    w18 = W1.reshape(F_IN, 8, 128).transpose(1, 0, 2)
    a1s8 = a1s.reshape(8, 128)
    a1d8 = a1d.reshape(8, 128)
    table1, esp, edp = _encode(x, w18, a1s8, a1d8)
    esed1 = jnp.concatenate([esp.reshape(4, 2, n).sum(1).ravel(),
                             edp.reshape(4, 2, n).sum(1).ravel()])
    acc1, ssum1p = _gat_layer(table1, esed1, src2, dst2, 8, HEADS, e)
    ssum1 = ssum1p.reshape(2, -1).sum(0)[:n * HEADS].reshape(n, HEADS)
    acc1 = acc1.reshape(8, 2, NPAD, 128).sum(1)[:, :n]
    out1 = acc1.transpose(1, 0, 2).reshape(n, HEADS, HID)
    out1 = out1 / (ssum1[:, :, None] + 1e-16)
    z = jax.nn.elu(out1.reshape(n, HEADS * HID) + b1)

    h2 = _matmul(z, W2)
    es2 = h2 @ a2s[0]
    ed2 = h2 @ a2d[0]
    esed2 = jnp.concatenate([es2, ed2])
    acc2, ssum2p = _gat_layer(h2[None], esed2, src2, dst2, 1, 1, e)
    ssum2 = ssum2p.reshape(2, -1).sum(0)[:n]
    acc2 = acc2.reshape(1, 2, NPAD, 128).sum(1)[0, :n]
    z2 = acc2 / (ssum2[:, None] + 1e-16) + b2

    row = edge_label_index[0].astype(jnp.int32)
    col = edge_label_index[1].astype(jnp.int32)
    zr, zc = _pair_gather(z2, row, col)
    bn_scale = 1.0 / jnp.sqrt(1.0 + 1e-5)
    out = _decoder(zr, zc, Wp1[:128], Wp1[128:], bp1, bn_scale * g1, be1,
                   Wp2, bp2, bn_scale * g2, be2, Wp3, bp3)
    return out[:, 0]
